# baseline (speedup 1.0000x reference)
import functools

import jax
import jax.numpy as jnp
from jax import lax
from jax.experimental import pallas as pl
from jax.experimental.pallas import tpu as pltpu
from jax.experimental.pallas import tpu_sc as plsc

_U_DIM = 100000
_Z_DIM = 64
_BATCH = 16384

_NC = 2
_NS = 16
_NW = _NC * _NS
_Q = 32768
_NQ = 4
_TAIL = _U_DIM - 3 * _Q
_TAIL_PAD = 1792
_UNROLL = 8

_mesh = plsc.VectorSubcoreMesh(core_axis_name="c", subcore_axis_name="s")


@functools.partial(
    pl.kernel,
    mesh=_mesh,
    compiler_params=pltpu.CompilerParams(needs_layout_passes=False),
    out_type=(
        jax.ShapeDtypeStruct((_Z_DIM, _BATCH), jnp.float32),
        jax.ShapeDtypeStruct((_Z_DIM, _BATCH), jnp.float32),
    ),
    scratch_types=[
        pltpu.VMEM((_BATCH,), jnp.int32),
        pltpu.VMEM((_Q,), jnp.float32),
        pltpu.VMEM((_Q,), jnp.float32),
        pltpu.VMEM((_BATCH,), jnp.float32),
        pltpu.VMEM((_BATCH,), jnp.float32),
        pltpu.SemaphoreType.DMA,
        pltpu.SemaphoreType.DMA,
        pltpu.SemaphoreType.DMA,
    ],
)
def _tgather(mt, lt, mtail, ltail, u_hbm, om, ol, u_v, qb0, qb1, outr0, outr1,
             semq0, semq1, osem):
    wid = lax.axis_index("s") * _NC + lax.axis_index("c")
    pltpu.sync_copy(u_hbm, u_v)
    qsems = (semq0, semq1)
    qbufs = (qb0, qb1)
    outrs = (outr0, outr1)
    d0 = wid * 2
    rows = [
        (mt, mtail, om, d0),
        (mt, mtail, om, d0 + 1),
        (lt, ltail, ol, d0),
        (lt, ltail, ol, d0 + 1),
    ]
    iota = jax.lax.broadcasted_iota(jnp.int32, (16,), 0)

    def start_q(r, q):
        src, tail_src, _, d = rows[r]
        if q < _NQ - 1:
            return pltpu.async_copy(
                src.at[d, pl.ds(q * _Q, _Q)],
                qbufs[q % 2].at[pl.ds(0, _Q)],
                qsems[q % 2],
            )
        return pltpu.async_copy(
            tail_src.at[d],
            qbufs[q % 2].at[pl.ds(0, _TAIL_PAD)],
            qsems[q % 2],
        )

    def sweep(r, q):
        outr = outrs[r % 2]
        qbuf = qbufs[q % 2]

        @plsc.parallel_loop(0, _BATCH, step=16 * _UNROLL)
        def body(i):
            for j in range(_UNROLL):
                base = i + j * 16
                u = u_v[pl.ds(base, 16)]
                m = (u >> 15) == q
                local = u & (_Q - 1)
                val = plsc.load_gather(qbuf, [local])
                plsc.store_scatter(outr, [base + iota], val, mask=m)

    tasks = [(r, q) for r in range(4) for q in range(_NQ)]
    out_pending = [None, None]
    cps = [None] * len(tasks)
    cps[0] = start_q(*tasks[0])
    cps[1] = start_q(*tasks[1])
    for k, (r, q) in enumerate(tasks):
        if q == 0 and out_pending[r % 2] is not None:
            out_pending[r % 2].wait()
            out_pending[r % 2] = None
        cps[k].wait()
        sweep(r, q)
        if k + 2 < len(tasks):
            cps[k + 2] = start_q(*tasks[k + 2])
        if q == _NQ - 1:
            _, _, dst, d = rows[r]
            out_pending[r % 2] = pltpu.async_copy(
                outrs[r % 2], dst.at[d], osem
            )
    for ob in range(2):
        if out_pending[ob] is not None:
            out_pending[ob].wait()


def kernel(u, embed_mean, embed_log_var):
    mt = embed_mean.T
    lt = embed_log_var.T
    pad = ((0, 0), (0, _TAIL_PAD - _TAIL))
    mtail = jnp.pad(mt[:, 3 * _Q:], pad)
    ltail = jnp.pad(lt[:, 3 * _Q:], pad)
    om, ol = _tgather(mt, lt, mtail, ltail, u.astype(jnp.int32))
    return om.T, ol.T
